# baseline (speedup 1.0000x reference)
import functools

import jax
import jax.numpy as jnp
from jax import lax
from jax.experimental import pallas as pl
from jax.experimental.pallas import tpu as pltpu
import jax.experimental.pallas.tpu_sc as plsc

H = 32
S = 2048
CM = 3968
FP = 4224
LAG = 16
NC = 2
NS = 16


def _prep_body(tab_ref, out_ref, flp_s, buf, sem):
    t = tab_ref[...]
    flp_s[:, :2 * S] = jnp.transpose(t)
    flp_s[:, 2 * S:] = jnp.zeros((H, FP - 2 * S), jnp.float32)
    for a in range(16):
        sl = a & 1
        if a >= 2:
            pltpu.make_async_copy(buf.at[sl], out_ref.at[:, pl.ds(0, 8), :], sem).wait()
        for b in range(8):
            buf[sl, :, b, :] = flp_s[:, a * 8 + (7 - b): a * 8 + (7 - b) + CM]
        pltpu.make_async_copy(buf.at[sl], out_ref.at[:, pl.ds(8 * a, 8), :], sem).start()
    for _ in range(2):
        pltpu.make_async_copy(buf.at[0], out_ref.at[:, pl.ds(0, 8), :], sem).wait()


def _prep(table):
    tfp = jnp.concatenate(
        [table[::-1], jnp.zeros((1, H), jnp.float32)], axis=0)
    return pl.pallas_call(
        _prep_body,
        out_specs=pl.BlockSpec(memory_space=pl.ANY),
        out_shape=jax.ShapeDtypeStruct((H, 128, CM), jnp.float32),
        scratch_shapes=[
            pltpu.VMEM((H, FP), jnp.float32),
            pltpu.VMEM((2, H, 8, CM), jnp.float32),
            pltpu.SemaphoreType.DMA,
        ],
    )(tfp)


def _sc_body(b_hbm, out_hbm, bq_v, sem, sem_load):
    cc = lax.axis_index("c")
    ss = lax.axis_index("s")
    w = cc * NS + ss
    t = jnp.bitwise_and(w, 3)
    hbase = lax.shift_right_logical(w, 2)

    def one_pass(p, carry):
        h = p * 8 + hbase
        for aq in range(4):
            row0 = pl.multiple_of((t * 4 + aq) * 8, 8)
            pltpu.make_async_copy(
                b_hbm.at[h, pl.ds(row0, 8), :], bq_v.at[aq], sem_load).start()
        n = 0
        for aq in range(4):
            pltpu.make_async_copy(
                b_hbm.at[h, pl.ds(0, 8), :], bq_v.at[aq], sem_load).wait()
            for j in range(16):
                i0 = pl.multiple_of(2040 - 32 * t - 8 * aq - 128 * j, 8)
                pltpu.make_async_copy(
                    bq_v.at[aq, :, pl.ds(128 * j, S)],
                    out_hbm.at[h, pl.ds(i0, 8), :],
                    sem,
                ).start()
                n += 1
                if n > LAG:
                    pltpu.make_async_copy(
                        bq_v.at[0, :, pl.ds(0, S)],
                        out_hbm.at[h, pl.ds(pl.multiple_of(2040 - 32 * t, 8), 8), :],
                        sem,
                    ).wait()
        for _ in range(min(LAG, 64)):
            pltpu.make_async_copy(
                bq_v.at[0, :, pl.ds(0, S)],
                out_hbm.at[h, pl.ds(pl.multiple_of(2040 - 32 * t, 8), 8), :],
                sem,
            ).wait()
        return carry

    lax.fori_loop(0, 4, one_pass, 0)


@functools.lru_cache(maxsize=1)
def _sc_call():
    return functools.partial(
        pl.kernel,
        out_type=jax.ShapeDtypeStruct((H, S, S), jnp.float32),
        mesh=plsc.VectorSubcoreMesh(
            core_axis_name="c", subcore_axis_name="s",
            num_cores=NC, num_subcores=NS),
        scratch_types=[
            pltpu.VMEM((4, 8, CM), jnp.float32),
            pltpu.SemaphoreType.DMA,
            pltpu.SemaphoreType.DMA,
        ],
    )(_sc_body)


def kernel(seq_len, relative_position_bias_table, relative_position_index):
    del seq_len, relative_position_index
    bank = _prep(relative_position_bias_table.astype(jnp.float32))
    return _sc_call()(bank)


# speedup vs baseline: 1.0151x; 1.0151x over previous
import functools

import jax
import jax.numpy as jnp
from jax import lax
from jax.experimental import pallas as pl
from jax.experimental.pallas import tpu as pltpu
import jax.experimental.pallas.tpu_sc as plsc

H = 32
S = 2048
CM = 3968
FP = 4224
LAG = 16
NC = 2
NS = 16


def _prep_body(tab_ref, out_ref, flp_s, buf, sem):
    t = tab_ref[...]
    flp_s[:, :2 * S] = jnp.transpose(t)
    flp_s[:, 2 * S:] = jnp.zeros((H, FP - 2 * S), jnp.float32)
    for a in range(16):
        sl = a % 3
        if a >= 3:
            pltpu.make_async_copy(buf.at[sl], out_ref.at[:, pl.ds(0, 8), :], sem).wait()
        for b in range(8):
            buf[sl, :, b, :] = flp_s[:, a * 8 + (7 - b): a * 8 + (7 - b) + CM]
        pltpu.make_async_copy(buf.at[sl], out_ref.at[:, pl.ds(8 * a, 8), :], sem).start()
    for _ in range(3):
        pltpu.make_async_copy(buf.at[0], out_ref.at[:, pl.ds(0, 8), :], sem).wait()


def _prep(table):
    tfp = jnp.concatenate(
        [table[::-1], jnp.zeros((1, H), jnp.float32)], axis=0)
    return pl.pallas_call(
        _prep_body,
        out_specs=pl.BlockSpec(memory_space=pl.ANY),
        out_shape=jax.ShapeDtypeStruct((H, 128, CM), jnp.float32),
        scratch_shapes=[
            pltpu.VMEM((H, FP), jnp.float32),
            pltpu.VMEM((3, H, 8, CM), jnp.float32),
            pltpu.SemaphoreType.DMA,
        ],
    )(tfp)


def _sc_body(b_hbm, out_hbm, bq_v, sem, sem_load):
    cc = lax.axis_index("c")
    ss = lax.axis_index("s")
    w = cc * NS + ss
    t = jnp.bitwise_and(w, 3)
    hbase = lax.shift_right_logical(w, 2)

    def one_pass(p, carry):
        h = p * 8 + hbase
        for aq in range(4):
            row0 = pl.multiple_of((t * 4 + aq) * 8, 8)
            pltpu.make_async_copy(
                b_hbm.at[h, pl.ds(row0, 8), :], bq_v.at[aq], sem_load).start()
        n = 0
        for aq in range(4):
            pltpu.make_async_copy(
                b_hbm.at[h, pl.ds(0, 8), :], bq_v.at[aq], sem_load).wait()
            for j in range(16):
                i0 = pl.multiple_of(2040 - 32 * t - 8 * aq - 128 * j, 8)
                pltpu.make_async_copy(
                    bq_v.at[aq, :, pl.ds(128 * j, S)],
                    out_hbm.at[h, pl.ds(i0, 8), :],
                    sem,
                ).start()
                n += 1
                if n > LAG:
                    pltpu.make_async_copy(
                        bq_v.at[0, :, pl.ds(0, S)],
                        out_hbm.at[h, pl.ds(pl.multiple_of(2040 - 32 * t, 8), 8), :],
                        sem,
                    ).wait()
        for _ in range(min(LAG, 64)):
            pltpu.make_async_copy(
                bq_v.at[0, :, pl.ds(0, S)],
                out_hbm.at[h, pl.ds(pl.multiple_of(2040 - 32 * t, 8), 8), :],
                sem,
            ).wait()
        return carry

    lax.fori_loop(0, 4, one_pass, 0)


@functools.lru_cache(maxsize=1)
def _sc_call():
    return functools.partial(
        pl.kernel,
        out_type=jax.ShapeDtypeStruct((H, S, S), jnp.float32),
        mesh=plsc.VectorSubcoreMesh(
            core_axis_name="c", subcore_axis_name="s",
            num_cores=NC, num_subcores=NS),
        scratch_types=[
            pltpu.VMEM((4, 8, CM), jnp.float32),
            pltpu.SemaphoreType.DMA,
            pltpu.SemaphoreType.DMA,
        ],
    )(_sc_body)


def kernel(seq_len, relative_position_bias_table, relative_position_index):
    del seq_len, relative_position_index
    bank = _prep(relative_position_bias_table.astype(jnp.float32))
    return _sc_call()(bank)
